# Initial kernel scaffold; baseline (speedup 1.0000x reference)
import jax
import jax.numpy as jnp
from jax.experimental import pallas as pl


def kernel(h, edge_index, W_in, W_out):
    raise NotImplementedError("write your pallas kernel here")


# speedup vs baseline: 8.7448x; 8.7448x over previous
import functools

import jax
import jax.numpy as jnp
from jax import lax
from jax.experimental import pallas as pl
from jax.experimental.pallas import tpu as pltpu
from jax.experimental.pallas import tpu_sc as plsc

_N = 10000
_E = 320000
_D = 128
_DA = 144
_NC = 2
_NS = 16
_NW = _NC * _NS
_EPW = _E // _NW
_C = 80
_NCH = _EPW // _C

_RPT = 640
_RLAST = _N - 15 * _RPT


def _sc_segment_sum(haug, src_t, dst_t, zeros):
    mesh = plsc.VectorSubcoreMesh(core_axis_name="c", subcore_axis_name="s")

    @functools.partial(
        pl.kernel,
        mesh=mesh,
        out_type=jax.ShapeDtypeStruct((_NC, _N, _DA), jnp.float32),
        scratch_types=[
            pltpu.VMEM((_NCH, _C), jnp.int32),
            pltpu.VMEM((_NCH, _C), jnp.int32),
            pltpu.VMEM((_C, _DA), jnp.float32),
            pltpu.VMEM_SHARED((_N, _DA), jnp.float32),
            pltpu.SemaphoreType.DMA,
        ],
        compiler_params=pltpu.CompilerParams(use_tc_tiling_on_sc=False),
    )
    def k(haug_hbm, src_hbm, dst_hbm, zero_hbm, out_hbm,
          src_v, dst_v, rows_v, acc, sem):
        cid = lax.axis_index("c")
        sid = lax.axis_index("s")
        wid = sid * _NC + cid

        @pl.when(sid < _NS - 1)
        def _():
            pltpu.sync_copy(zero_hbm.at[pl.ds(sid * _RPT, _RPT)],
                            acc.at[pl.ds(sid * _RPT, _RPT)])

        @pl.when(sid == _NS - 1)
        def _():
            pltpu.sync_copy(zero_hbm.at[pl.ds(15 * _RPT, _RLAST)],
                            acc.at[pl.ds(15 * _RPT, _RLAST)])

        pltpu.sync_copy(src_hbm.at[wid], src_v)
        pltpu.sync_copy(dst_hbm.at[wid], dst_v)
        plsc.subcore_barrier()

        def body(j, carry):
            pltpu.async_copy(haug_hbm.at[src_v.at[j]], rows_v, sem).wait()
            pltpu.sync_copy(rows_v, acc.at[dst_v.at[j]], add=True)
            return carry

        lax.fori_loop(0, _NCH, body, None)

        plsc.subcore_barrier()

        @pl.when(sid < _NS - 1)
        def _():
            pltpu.sync_copy(acc.at[pl.ds(sid * _RPT, _RPT)],
                            out_hbm.at[cid, pl.ds(sid * _RPT, _RPT)])

        @pl.when(sid == _NS - 1)
        def _():
            pltpu.sync_copy(acc.at[pl.ds(15 * _RPT, _RLAST)],
                            out_hbm.at[cid, pl.ds(15 * _RPT, _RLAST)])

    return k(haug, src_t, dst_t, zeros)


def _tc_combine(sp, h, wpad_in, w_out_t):
    blk = 1000

    def body(sp_ref, h_ref, wi_ref, wo_ref, o_ref):
        s = sp_ref[0] + sp_ref[1]
        deg = s[:, _D:_D + 1]
        y_in = jnp.dot(s, wi_ref[...], preferred_element_type=jnp.float32)
        y_out = jnp.dot(h_ref[...], wo_ref[...], preferred_element_type=jnp.float32)
        o_ref[...] = y_in + deg * y_out

    return pl.pallas_call(
        body,
        grid=(_N // blk,),
        in_specs=[
            pl.BlockSpec((_NC, blk, _DA), lambda i: (0, i, 0)),
            pl.BlockSpec((blk, _D), lambda i: (i, 0)),
            pl.BlockSpec((_DA, _D), lambda i: (0, 0)),
            pl.BlockSpec((_D, _D), lambda i: (0, 0)),
        ],
        out_specs=pl.BlockSpec((blk, _D), lambda i: (i, 0)),
        out_shape=jax.ShapeDtypeStruct((_N, _D), jnp.float32),
    )(sp, h, wpad_in, w_out_t)


def kernel(h, edge_index, W_in, W_out):
    n, d = h.shape
    haug = jnp.zeros((n, _DA), jnp.float32).at[:, :d].set(h).at[:, d].set(1.0)
    src = edge_index[0].reshape(_NW, _NCH, _C)
    dst = edge_index[1].reshape(_NW, _NCH, _C)
    zeros = jnp.zeros((n, _DA), jnp.float32)
    sp = _sc_segment_sum(haug, src, dst, zeros)
    wpad_in = jnp.zeros((_DA, d), jnp.float32).at[:d].set(W_in.T)
    return _tc_combine(sp, h, wpad_in, W_out.T)


# speedup vs baseline: 10.5440x; 1.2057x over previous
import functools

import jax
import jax.numpy as jnp
from jax import lax
from jax.experimental import pallas as pl
from jax.experimental.pallas import tpu as pltpu
from jax.experimental.pallas import tpu_sc as plsc

_N = 10000
_E = 320000
_D = 128
_DA = 144
_NC = 2
_NS = 16
_NW = _NC * _NS
_EPW = _E // _NW
_C = 40
_NCH = _EPW // _C

_RPT = 640
_RLAST = _N - 15 * _RPT


def _sc_segment_sum(haug, src_t, dst_t, zeros):
    mesh = plsc.VectorSubcoreMesh(core_axis_name="c", subcore_axis_name="s")

    @functools.partial(
        pl.kernel,
        mesh=mesh,
        out_type=jax.ShapeDtypeStruct((_NC, _N, _DA), jnp.float32),
        scratch_types=[
            pltpu.VMEM((_NCH, _C), jnp.int32),
            pltpu.VMEM((_NCH, _C), jnp.int32),
            pltpu.VMEM((_C, _DA), jnp.float32),
            pltpu.VMEM((_C, _DA), jnp.float32),
            pltpu.VMEM_SHARED((_N, _DA), jnp.float32),
            pltpu.SemaphoreType.DMA,
            pltpu.SemaphoreType.DMA,
        ],
        compiler_params=pltpu.CompilerParams(use_tc_tiling_on_sc=False),
    )
    def k(haug_hbm, src_hbm, dst_hbm, zero_hbm, out_hbm,
          src_v, dst_v, rows0, rows1, acc, sem0, sem1):
        cid = lax.axis_index("c")
        sid = lax.axis_index("s")
        wid = sid * _NC + cid

        @pl.when(sid < _NS - 1)
        def _():
            pltpu.sync_copy(zero_hbm.at[pl.ds(sid * _RPT, _RPT)],
                            acc.at[pl.ds(sid * _RPT, _RPT)])

        @pl.when(sid == _NS - 1)
        def _():
            pltpu.sync_copy(zero_hbm.at[pl.ds(15 * _RPT, _RLAST)],
                            acc.at[pl.ds(15 * _RPT, _RLAST)])

        pltpu.sync_copy(src_hbm.at[wid], src_v)
        pltpu.sync_copy(dst_hbm.at[wid], dst_v)
        plsc.subcore_barrier()

        def start(j, buf, sem):
            pltpu.async_copy(haug_hbm.at[src_v.at[j]], buf, sem)

        def finish(j, buf, sem):
            pltpu.make_async_copy(haug_hbm.at[src_v.at[j]], buf, sem).wait()
            pltpu.sync_copy(buf, acc.at[dst_v.at[j]], add=True)

        start(0, rows0, sem0)
        start(1, rows1, sem1)

        def body(jj, carry):
            j0 = 2 * jj
            finish(j0, rows0, sem0)

            @pl.when(j0 + 2 < _NCH)
            def _():
                start(j0 + 2, rows0, sem0)

            finish(j0 + 1, rows1, sem1)

            @pl.when(j0 + 3 < _NCH)
            def _():
                start(j0 + 3, rows1, sem1)

            return carry

        lax.fori_loop(0, _NCH // 2, body, None)

        plsc.subcore_barrier()

        @pl.when(sid < _NS - 1)
        def _():
            pltpu.sync_copy(acc.at[pl.ds(sid * _RPT, _RPT)],
                            out_hbm.at[cid, pl.ds(sid * _RPT, _RPT)])

        @pl.when(sid == _NS - 1)
        def _():
            pltpu.sync_copy(acc.at[pl.ds(15 * _RPT, _RLAST)],
                            out_hbm.at[cid, pl.ds(15 * _RPT, _RLAST)])

    return k(haug, src_t, dst_t, zeros)


def _tc_combine(sp, h, wpad_in, w_out_t):
    blk = 1000

    def body(sp_ref, h_ref, wi_ref, wo_ref, o_ref):
        s = sp_ref[0] + sp_ref[1]
        deg = s[:, _D:_D + 1]
        y_in = jnp.dot(s, wi_ref[...], preferred_element_type=jnp.float32)
        y_out = jnp.dot(h_ref[...], wo_ref[...], preferred_element_type=jnp.float32)
        o_ref[...] = y_in + deg * y_out

    return pl.pallas_call(
        body,
        grid=(_N // blk,),
        in_specs=[
            pl.BlockSpec((_NC, blk, _DA), lambda i: (0, i, 0)),
            pl.BlockSpec((blk, _D), lambda i: (i, 0)),
            pl.BlockSpec((_DA, _D), lambda i: (0, 0)),
            pl.BlockSpec((_D, _D), lambda i: (0, 0)),
        ],
        out_specs=pl.BlockSpec((blk, _D), lambda i: (i, 0)),
        out_shape=jax.ShapeDtypeStruct((_N, _D), jnp.float32),
    )(sp, h, wpad_in, w_out_t)


def kernel(h, edge_index, W_in, W_out):
    n, d = h.shape
    haug = jnp.zeros((n, _DA), jnp.float32).at[:, :d].set(h).at[:, d].set(1.0)
    src = edge_index[0].reshape(_NW, _NCH, _C)
    dst = edge_index[1].reshape(_NW, _NCH, _C)
    zeros = jnp.zeros((n, _DA), jnp.float32)
    sp = _sc_segment_sum(haug, src, dst, zeros)
    wpad_in = jnp.zeros((_DA, d), jnp.float32).at[:d].set(W_in.T)
    return _tc_combine(sp, h, wpad_in, W_out.T)


# speedup vs baseline: 12.3645x; 1.1726x over previous
import functools

import jax
import jax.numpy as jnp
from jax import lax
from jax.experimental import pallas as pl
from jax.experimental.pallas import tpu as pltpu
from jax.experimental.pallas import tpu_sc as plsc

_N = 10000
_E = 320000
_D = 128
_DG = 16
_NC = 2
_NS = 16
_NW = _NC * _NS
_EPW = _E // _NW
_C = 40
_NCH = _EPW // _C

_RPT = 640
_RLAST = _N - 15 * _RPT


def _sc_segment_sum(h, src_t, dst_t, zeros, zeros_dg, ones):
    mesh = plsc.VectorSubcoreMesh(core_axis_name="c", subcore_axis_name="s")

    @functools.partial(
        pl.kernel,
        mesh=mesh,
        out_type=[
            jax.ShapeDtypeStruct((_NC, _N, _D), jnp.float32),
            jax.ShapeDtypeStruct((_NC, _N, _DG), jnp.float32),
        ],
        scratch_types=[
            pltpu.VMEM((_NCH, _C), jnp.int32),
            pltpu.VMEM((_NCH, _C), jnp.int32),
            pltpu.VMEM((_C, _D), jnp.float32),
            pltpu.VMEM((_C, _D), jnp.float32),
            pltpu.VMEM((_C, _DG), jnp.float32),
            pltpu.VMEM_SHARED((_N, _D), jnp.float32),
            pltpu.VMEM_SHARED((_N, _DG), jnp.float32),
            pltpu.SemaphoreType.DMA,
            pltpu.SemaphoreType.DMA,
        ],
        compiler_params=pltpu.CompilerParams(use_tc_tiling_on_sc=False),
    )
    def k(h_hbm, src_hbm, dst_hbm, zero_hbm, zerodg_hbm, ones_hbm,
          out_hbm, outdg_hbm,
          src_v, dst_v, rows0, rows1, ones_v, acc, dacc, sem0, sem1):
        cid = lax.axis_index("c")
        sid = lax.axis_index("s")
        wid = sid * _NC + cid

        @pl.when(sid < _NS - 1)
        def _():
            pltpu.sync_copy(zero_hbm.at[pl.ds(sid * _RPT, _RPT)],
                            acc.at[pl.ds(sid * _RPT, _RPT)])
            pltpu.sync_copy(zerodg_hbm.at[pl.ds(sid * _RPT, _RPT)],
                            dacc.at[pl.ds(sid * _RPT, _RPT)])

        @pl.when(sid == _NS - 1)
        def _():
            pltpu.sync_copy(zero_hbm.at[pl.ds(15 * _RPT, _RLAST)],
                            acc.at[pl.ds(15 * _RPT, _RLAST)])
            pltpu.sync_copy(zerodg_hbm.at[pl.ds(15 * _RPT, _RLAST)],
                            dacc.at[pl.ds(15 * _RPT, _RLAST)])

        pltpu.sync_copy(src_hbm.at[wid], src_v)
        pltpu.sync_copy(dst_hbm.at[wid], dst_v)
        pltpu.sync_copy(ones_hbm, ones_v)
        plsc.subcore_barrier()

        def start(j, buf, sem):
            pltpu.async_copy(h_hbm.at[src_v.at[j]], buf, sem)

        def finish(j, buf, sem):
            pltpu.make_async_copy(h_hbm.at[src_v.at[j]], buf, sem).wait()
            pltpu.sync_copy(buf, acc.at[dst_v.at[j]], add=True)
            pltpu.sync_copy(ones_v, dacc.at[dst_v.at[j]], add=True)

        start(0, rows0, sem0)
        start(1, rows1, sem1)

        def body(jj, carry):
            j0 = 2 * jj
            finish(j0, rows0, sem0)

            @pl.when(j0 + 2 < _NCH)
            def _():
                start(j0 + 2, rows0, sem0)

            finish(j0 + 1, rows1, sem1)

            @pl.when(j0 + 3 < _NCH)
            def _():
                start(j0 + 3, rows1, sem1)

            return carry

        lax.fori_loop(0, _NCH // 2, body, None)

        plsc.subcore_barrier()

        @pl.when(sid < _NS - 1)
        def _():
            pltpu.sync_copy(acc.at[pl.ds(sid * _RPT, _RPT)],
                            out_hbm.at[cid, pl.ds(sid * _RPT, _RPT)])
            pltpu.sync_copy(dacc.at[pl.ds(sid * _RPT, _RPT)],
                            outdg_hbm.at[cid, pl.ds(sid * _RPT, _RPT)])

        @pl.when(sid == _NS - 1)
        def _():
            pltpu.sync_copy(acc.at[pl.ds(15 * _RPT, _RLAST)],
                            out_hbm.at[cid, pl.ds(15 * _RPT, _RLAST)])
            pltpu.sync_copy(dacc.at[pl.ds(15 * _RPT, _RLAST)],
                            outdg_hbm.at[cid, pl.ds(15 * _RPT, _RLAST)])

    return k(h, src_t, dst_t, zeros, zeros_dg, ones)


def _tc_combine(sp, dp, h, w_in_t, w_out_t):
    blk = 1000

    def body(sp_ref, dp_ref, h_ref, wi_ref, wo_ref, o_ref):
        s = sp_ref[0] + sp_ref[1]
        deg = dp_ref[0, :, :1] + dp_ref[1, :, :1]
        y_in = jnp.dot(s, wi_ref[...], preferred_element_type=jnp.float32)
        y_out = jnp.dot(h_ref[...], wo_ref[...], preferred_element_type=jnp.float32)
        o_ref[...] = y_in + deg * y_out

    return pl.pallas_call(
        body,
        grid=(_N // blk,),
        in_specs=[
            pl.BlockSpec((_NC, blk, _D), lambda i: (0, i, 0)),
            pl.BlockSpec((_NC, blk, _DG), lambda i: (0, i, 0)),
            pl.BlockSpec((blk, _D), lambda i: (i, 0)),
            pl.BlockSpec((_D, _D), lambda i: (0, 0)),
            pl.BlockSpec((_D, _D), lambda i: (0, 0)),
        ],
        out_specs=pl.BlockSpec((blk, _D), lambda i: (i, 0)),
        out_shape=jax.ShapeDtypeStruct((_N, _D), jnp.float32),
    )(sp, dp, h, w_in_t, w_out_t)


def kernel(h, edge_index, W_in, W_out):
    n, d = h.shape
    src = edge_index[0].reshape(_NW, _NCH, _C)
    dst = edge_index[1].reshape(_NW, _NCH, _C)
    zeros = jnp.zeros((n, _D), jnp.float32)
    zeros_dg = jnp.zeros((n, _DG), jnp.float32)
    ones = jnp.ones((_C, _DG), jnp.float32)
    sp, dp = _sc_segment_sum(h, src, dst, zeros, zeros_dg, ones)
    return _tc_combine(sp, dp, h, W_in.T, W_out.T)


# speedup vs baseline: 14.7758x; 1.1950x over previous
import functools

import jax
import jax.numpy as jnp
from jax import lax
from jax.experimental import pallas as pl
from jax.experimental.pallas import tpu as pltpu
from jax.experimental.pallas import tpu_sc as plsc

_N = 10000
_E = 320000
_D = 128
_DG = 16
_NC = 2
_NS = 16
_NW = _NC * _NS
_EPW = _E // _NW
_C = 40
_NCH = _EPW // _C

_RPT = 640
_RLAST = _N - 15 * _RPT


def _sc_segment_sum(h, src_t, dst_t, zeros, zeros_dg, ones):
    mesh = plsc.VectorSubcoreMesh(core_axis_name="c", subcore_axis_name="s")

    @functools.partial(
        pl.kernel,
        mesh=mesh,
        out_type=[
            jax.ShapeDtypeStruct((_NC, _N, _D), jnp.float32),
            jax.ShapeDtypeStruct((_NC, _N, _DG), jnp.float32),
        ],
        scratch_types=[
            pltpu.VMEM((_NCH, _C), jnp.int32),
            pltpu.VMEM((_NCH, _C), jnp.int32),
            pltpu.VMEM((_C, _D), jnp.float32),
            pltpu.VMEM((_C, _D), jnp.float32),
            pltpu.VMEM((_C, _D), jnp.float32),
            pltpu.VMEM((_C, _DG), jnp.float32),
            pltpu.VMEM_SHARED((_N, _D), jnp.float32),
            pltpu.VMEM_SHARED((_N, _DG), jnp.float32),
            pltpu.SemaphoreType.DMA,
            pltpu.SemaphoreType.DMA,
            pltpu.SemaphoreType.DMA,
            pltpu.SemaphoreType.DMA,
            pltpu.SemaphoreType.DMA,
            pltpu.SemaphoreType.DMA,
            pltpu.SemaphoreType.DMA,
            pltpu.SemaphoreType.DMA,
            pltpu.SemaphoreType.DMA,
        ],
        compiler_params=pltpu.CompilerParams(use_tc_tiling_on_sc=False),
    )
    def k(h_hbm, src_hbm, dst_hbm, zero_hbm, zerodg_hbm, ones_hbm,
          out_hbm, outdg_hbm,
          src_v, dst_v, rows0, rows1, rows2, ones_v, acc, dacc,
          g0, g1, g2, s0, s1, s2, o0, o1, o2):
        cid = lax.axis_index("c")
        sid = lax.axis_index("s")
        wid = sid * _NC + cid

        rows = (rows0, rows1, rows2)
        gs = (g0, g1, g2)
        ss = (s0, s1, s2)
        os_ = (o0, o1, o2)

        @pl.when(sid < _NS - 1)
        def _():
            pltpu.sync_copy(zero_hbm.at[pl.ds(sid * _RPT, _RPT)],
                            acc.at[pl.ds(sid * _RPT, _RPT)])
            pltpu.sync_copy(zerodg_hbm.at[pl.ds(sid * _RPT, _RPT)],
                            dacc.at[pl.ds(sid * _RPT, _RPT)])

        @pl.when(sid == _NS - 1)
        def _():
            pltpu.sync_copy(zero_hbm.at[pl.ds(15 * _RPT, _RLAST)],
                            acc.at[pl.ds(15 * _RPT, _RLAST)])
            pltpu.sync_copy(zerodg_hbm.at[pl.ds(15 * _RPT, _RLAST)],
                            dacc.at[pl.ds(15 * _RPT, _RLAST)])

        pltpu.sync_copy(src_hbm.at[wid], src_v)
        pltpu.sync_copy(dst_hbm.at[wid], dst_v)
        pltpu.sync_copy(ones_hbm, ones_v)
        plsc.subcore_barrier()

        def start(j, p):
            pltpu.async_copy(h_hbm.at[src_v.at[j]], rows[p], gs[p])

        def finish(j, p, q, first=False):
            pltpu.make_async_copy(h_hbm.at[src_v.at[j]], rows[p], gs[p]).wait()
            pltpu.async_copy(rows[p], acc.at[dst_v.at[j]], ss[p], add=True)
            pltpu.async_copy(ones_v, dacc.at[dst_v.at[j]], os_[p], add=True)
            if not first:
                pltpu.make_async_copy(rows[q], acc.at[dst_v.at[j - 1]],
                                      ss[q]).wait()
                pltpu.make_async_copy(ones_v, dacc.at[dst_v.at[j - 1]],
                                      os_[q]).wait()

            @pl.when(j + 2 < _NCH)
            def _():
                start(j + 2, q)

        start(0, 0)
        start(1, 1)
        finish(0, 0, 2, first=True)

        def body(i, carry):
            a = 3 * i + 1
            finish(a, 1, 0)
            finish(a + 1, 2, 1)
            finish(a + 2, 0, 2)
            return carry

        lax.fori_loop(0, (_NCH - 1) // 3, body, None)
        pltpu.make_async_copy(rows[0], acc.at[dst_v.at[_NCH - 1]], ss[0]).wait()
        pltpu.make_async_copy(ones_v, dacc.at[dst_v.at[_NCH - 1]], os_[0]).wait()

        plsc.subcore_barrier()

        @pl.when(sid < _NS - 1)
        def _():
            pltpu.sync_copy(acc.at[pl.ds(sid * _RPT, _RPT)],
                            out_hbm.at[cid, pl.ds(sid * _RPT, _RPT)])
            pltpu.sync_copy(dacc.at[pl.ds(sid * _RPT, _RPT)],
                            outdg_hbm.at[cid, pl.ds(sid * _RPT, _RPT)])

        @pl.when(sid == _NS - 1)
        def _():
            pltpu.sync_copy(acc.at[pl.ds(15 * _RPT, _RLAST)],
                            out_hbm.at[cid, pl.ds(15 * _RPT, _RLAST)])
            pltpu.sync_copy(dacc.at[pl.ds(15 * _RPT, _RLAST)],
                            outdg_hbm.at[cid, pl.ds(15 * _RPT, _RLAST)])

    return k(h, src_t, dst_t, zeros, zeros_dg, ones)


def _tc_combine(sp, dp, h, w_in_t, w_out_t):
    blk = 1000

    def body(sp_ref, dp_ref, h_ref, wi_ref, wo_ref, o_ref):
        s = sp_ref[0] + sp_ref[1]
        deg = dp_ref[0, :, :1] + dp_ref[1, :, :1]
        y_in = jnp.dot(s, wi_ref[...], preferred_element_type=jnp.float32)
        y_out = jnp.dot(h_ref[...], wo_ref[...], preferred_element_type=jnp.float32)
        o_ref[...] = y_in + deg * y_out

    return pl.pallas_call(
        body,
        grid=(_N // blk,),
        in_specs=[
            pl.BlockSpec((_NC, blk, _D), lambda i: (0, i, 0)),
            pl.BlockSpec((_NC, blk, _DG), lambda i: (0, i, 0)),
            pl.BlockSpec((blk, _D), lambda i: (i, 0)),
            pl.BlockSpec((_D, _D), lambda i: (0, 0)),
            pl.BlockSpec((_D, _D), lambda i: (0, 0)),
        ],
        out_specs=pl.BlockSpec((blk, _D), lambda i: (i, 0)),
        out_shape=jax.ShapeDtypeStruct((_N, _D), jnp.float32),
    )(sp, dp, h, w_in_t, w_out_t)


def kernel(h, edge_index, W_in, W_out):
    n, d = h.shape
    src = edge_index[0].reshape(_NW, _NCH, _C)
    dst = edge_index[1].reshape(_NW, _NCH, _C)
    zeros = jnp.zeros((n, _D), jnp.float32)
    zeros_dg = jnp.zeros((n, _DG), jnp.float32)
    ones = jnp.ones((_C, _DG), jnp.float32)
    sp, dp = _sc_segment_sum(h, src, dst, zeros, zeros_dg, ones)
    return _tc_combine(sp, dp, h, W_in.T, W_out.T)
